# Initial kernel scaffold; baseline (speedup 1.0000x reference)
import jax
import jax.numpy as jnp
from jax.experimental import pallas as pl


def kernel(x, edge_index):
    raise NotImplementedError("write your pallas kernel here")


# speedup vs baseline: 16.7543x; 16.7543x over previous
import numpy as np

import jax
import jax.numpy as jnp
from jax import lax
from jax.experimental import pallas as pl
from jax.experimental.pallas import tpu as pltpu
from jax.experimental.pallas import tpu_sc as plsc

N = 10000
D = 128
E = 320000
NC = 2
NS = 16
L = 16
DH = D // NC
EPT = E // NS
CH = 80
NCHUNK = EPT // CH
NB = 400
NBLK = N // NB
TAB = E + 8

with np.errstate(divide="ignore"):
  _RTAB = (np.arange(TAB, dtype=np.float32) ** np.float32(-0.5)).astype(
      np.float32)


def _body(xh, rowh, colh, rtabh, outh, tblh,
          rix, cix, u_t, gA, gB, uv, nblk, sv, ib,
          deg_s, s_s, u_s, acc_s, semA, semB):
  c = lax.axis_index("c")
  s = lax.axis_index("s")
  cN = (c * N).astype(jnp.int32)

  zero16 = jnp.zeros((L,), jnp.float32)
  one16 = jnp.ones((L,), jnp.float32)

  pltpu.sync_copy(rowh.at[s], rix)
  pltpu.sync_copy(colh.at[s], cix)

  def _zero_rows(ref, nrows):
    def zr(r, carry):
      for j in range(DH // L):
        ref[r, pl.ds(j * L, L)] = zero16
      return carry
    lax.fori_loop(0, nrows, zr, 0)

  _zero_rows(nblk, NB)

  def zsv(i, carry):
    sv[pl.ds(i * L, L)] = zero16
    return carry
  lax.fori_loop(0, NB // L, zsv, 0)

  def ouv(i, carry):
    uv[pl.ds(i * L, L)] = one16
    return carry
  lax.fori_loop(0, CH // L, ouv, 0)

  def _for_owned_blocks(fn):
    def blk(b, carry):
      @pl.when(lax.rem(b, NS) == s)
      def _():
        fn(b)
      return carry
    lax.fori_loop(0, NBLK, blk, 0)

  def zshared(b):
    pltpu.sync_copy(nblk, acc_s.at[pl.ds(b * NB, NB)])
    pltpu.sync_copy(sv, deg_s.at[pl.ds(b * NB, NB)])
    pltpu.sync_copy(sv, s_s.at[pl.ds(b * NB, NB)])
  _for_owned_blocks(zshared)
  plsc.subcore_barrier()

  def degk(k, carry):
    pltpu.sync_copy(uv, deg_s.at[cix.at[k]], add=True)
    return carry
  lax.fori_loop(0, NCHUNK, degk, 0)
  plsc.subcore_barrier()

  def ublk(b):
    pltpu.sync_copy(deg_s.at[pl.ds(b * NB, NB)], sv)
    def urow(i, carry):
      ib[pl.ds(i * L, L)] = sv[pl.ds(i * L, L)].astype(jnp.int32)
      return carry
    lax.fori_loop(0, NB // L, urow, 0)
    for t in range(NB // CH):
      pltpu.sync_copy(rtabh.at[ib.at[pl.ds(t * CH, CH)]],
                      sv.at[pl.ds(t * CH, CH)])
    pltpu.sync_copy(sv, u_s.at[pl.ds(b * NB, NB)])
  _for_owned_blocks(ublk)
  plsc.subcore_barrier()

  pltpu.sync_copy(u_s, u_t)

  def sk(k, carry):
    def gi(i, c2):
      idx = rix[k, pl.ds(i * L, L)]
      uv[pl.ds(i * L, L)] = plsc.load_gather(u_t, [idx])
      return c2
    lax.fori_loop(0, CH // L, gi, 0)
    pltpu.sync_copy(uv, s_s.at[cix.at[k]], add=True)
    return carry
  lax.fori_loop(0, NCHUNK, sk, 0)

  def adjk(k, carry):
    def a2(i, c2):
      rix[k, pl.ds(i * L, L)] = rix[k, pl.ds(i * L, L)] + cN
      return c2
    lax.fori_loop(0, CH // L, a2, 0)
    return carry
  lax.fori_loop(0, NCHUNK, adjk, 0)
  plsc.subcore_barrier()

  def _scale_rows(b, get_scale):
    def srow(r, carry):
      sc = get_scale(b, r)
      for j in range(DH // L):
        nblk[r, pl.ds(j * L, L)] = nblk[r, pl.ds(j * L, L)] * sc
      return carry
    lax.fori_loop(0, NB, srow, 0)

  def _u_scale(b, r):
    gidx = jnp.full((L,), b * NB + r, jnp.int32)
    return plsc.load_gather(u_t, [gidx])

  def fblk(b):
    pltpu.sync_copy(xh.at[c, pl.ds(b * NB, NB)], nblk)
    _scale_rows(b, _u_scale)
    pltpu.sync_copy(nblk, tblh.at[pl.ds(cN + b * NB, NB)])
  _for_owned_blocks(fblk)
  plsc.subcore_barrier()

  def edge_pass():
    def ep(g, carry):
      k0 = 2 * g
      k1 = 2 * g + 1
      dA = pltpu.async_copy(tblh.at[rix.at[k0]], gA, semA)
      dB = pltpu.async_copy(tblh.at[rix.at[k1]], gB, semB)
      dA.wait()
      pltpu.sync_copy(gA, acc_s.at[cix.at[k0]], add=True)
      dB.wait()
      pltpu.sync_copy(gB, acc_s.at[cix.at[k1]], add=True)
      return carry
    lax.fori_loop(0, NCHUNK // 2, ep, 0)

  edge_pass()
  plsc.subcore_barrier()

  _zero_rows(gA, CH)

  def hblk(b):
    pltpu.sync_copy(acc_s.at[pl.ds(b * NB, NB)], nblk)
    pltpu.sync_copy(s_s.at[pl.ds(b * NB, NB)], sv)
    def us_scale(b2, r):
      uu = _u_scale(b2, r)
      ss = plsc.load_gather(sv, [jnp.full((L,), r, jnp.int32)])
      return jnp.where(ss > 0.0, uu / ss, 0.0)
    _scale_rows(b, us_scale)
    pltpu.sync_copy(nblk, tblh.at[pl.ds(cN + b * NB, NB)])
    for t in range(NB // CH):
      pltpu.sync_copy(gA, acc_s.at[pl.ds(b * NB + t * CH, CH)])
  _for_owned_blocks(hblk)
  plsc.subcore_barrier()

  edge_pass()
  plsc.subcore_barrier()

  def kblk(b):
    pltpu.sync_copy(acc_s.at[pl.ds(b * NB, NB)], nblk)
    pltpu.sync_copy(s_s.at[pl.ds(b * NB, NB)], sv)
    def inv_s(b2, r):
      ss = plsc.load_gather(sv, [jnp.full((L,), r, jnp.int32)])
      return jnp.where(ss > 0.0, 1.0 / ss, 0.0)
    _scale_rows(b, inv_s)
    pltpu.sync_copy(nblk, outh.at[c, pl.ds(b * NB, NB)])
  _for_owned_blocks(kblk)


_mesh = plsc.VectorSubcoreMesh(
    core_axis_name="c", subcore_axis_name="s", num_cores=NC, num_subcores=NS)

_gcn2 = pl.kernel(
    _body,
    out_type=[
        jax.ShapeDtypeStruct((NC, N, DH), jnp.float32),
        jax.ShapeDtypeStruct((NC * N, DH), jnp.float32),
    ],
    mesh=_mesh,
    compiler_params=pltpu.CompilerParams(needs_layout_passes=False, use_tc_tiling_on_sc=False),
    scratch_types=[
        pltpu.VMEM((NCHUNK, CH), jnp.int32),
        pltpu.VMEM((NCHUNK, CH), jnp.int32),
        pltpu.VMEM((N,), jnp.float32),
        pltpu.VMEM((CH, DH), jnp.float32),
        pltpu.VMEM((CH, DH), jnp.float32),
        pltpu.VMEM((CH,), jnp.float32),
        pltpu.VMEM((NB, DH), jnp.float32),
        pltpu.VMEM((NB,), jnp.float32),
        pltpu.VMEM((NB,), jnp.int32),
        pltpu.VMEM_SHARED((N,), jnp.float32),
        pltpu.VMEM_SHARED((N,), jnp.float32),
        pltpu.VMEM_SHARED((N,), jnp.float32),
        pltpu.VMEM_SHARED((N, DH), jnp.float32),
        pltpu.SemaphoreType.DMA,
        pltpu.SemaphoreType.DMA,
    ],
)


@jax.jit
def kernel(x, edge_index):
  ei = edge_index.astype(jnp.int32)
  row3 = ei[0].reshape(NS, NCHUNK, CH)
  col3 = ei[1].reshape(NS, NCHUNK, CH)
  xhalves = jnp.stack([x[:, :DH], x[:, DH:]])
  outh, _ = _gcn2(xhalves, row3, col3, jnp.asarray(_RTAB))
  return jnp.concatenate([outh[0], outh[1]], axis=1)
